# baseline (speedup 1.0000x reference)
import functools

import jax
import jax.numpy as jnp
from jax import lax
from jax.experimental import pallas as pl
from jax.experimental.pallas import tpu as pltpu
from jax.experimental.pallas import tpu_sc as plsc

_LANES = 16
_NUM_CORES = 2
_NUM_SUBCORES = 16
_NUM_WORKERS = _NUM_CORES * _NUM_SUBCORES
_CHUNK = 128
_NVAR = 8


@functools.lru_cache(maxsize=None)
def _build_sc_kernel(S: int, num_buckets: int, num_heads: int):
    L = _LANES
    NW = _NUM_WORKERS
    rows_total = num_heads * S
    assert rows_total % NW == 0
    rows_per_worker = rows_total // NW
    assert rows_per_worker % 64 == 0 and S % rows_per_worker == 0
    assert S % 128 == 0
    P = 2 * S
    assert (_NVAR * P) % (8 * _CHUNK) == 0 and P % _CHUNK == 0
    md = num_buckets // 2
    shift = md - (S - 1)

    mesh = plsc.VectorSubcoreMesh(core_axis_name="c", subcore_axis_name="s")

    @functools.partial(
        pl.kernel,
        mesh=mesh,
        out_type=jax.ShapeDtypeStruct((rows_total // 8, S // 128, 8, 128), jnp.float32),
        compiler_params=pltpu.CompilerParams(use_tc_tiling_on_sc=False),
        scratch_types=[
            pltpu.VMEM((_NVAR * P,), jnp.int32),
            pltpu.VMEM((_NVAR, P), jnp.float32),
            pltpu.SemaphoreType.DMA,
        ],
    )
    def sc_kernel(bt_hbm, out_hbm, idx_v, pw_v, sem):
        wid = lax.axis_index("s") * _NUM_CORES + lax.axis_index("c")
        row0 = wid * rows_per_worker
        h = row0 // S
        i0 = row0 - h * S

        iota = lax.iota(jnp.int32, L)
        hbase = h * num_buckets

        def build_idx(g, _):
            for s in range(2):
                base_u = (g * 2 + s) * L
                c0 = (base_u + shift) + iota
                for k in range(_NVAR):
                    idx_v[pl.ds(k * P + base_u, L)] = hbase + jnp.clip(
                        c0 + (_NVAR - 1 - k), 0, num_buckets - 1
                    )
            return 0

        lax.fori_loop(0, P // L // 2, build_idx, 0, unroll=False)

        n_chunks = _NVAR * P // _CHUNK
        per_row = P // _CHUNK

        def fire_chunk(c):
            k = c // per_row
            off = (c - k * per_row) * _CHUNK
            src = bt_hbm.at[idx_v.at[pl.ds(k * P + off, _CHUNK)]]
            return pltpu.async_copy(src, pw_v.at[k, pl.ds(off, _CHUNK)], sem)

        GDEPTH = 64
        GB = 8
        for b in range(GDEPTH):
            fire_chunk(b)

        def gather(g, _):
            cb = GDEPTH + g * GB
            handles = [fire_chunk(cb + b) for b in range(GB)]
            for hd in handles:
                hd.wait()
            return 0

        lax.fori_loop(0, (n_chunks - GDEPTH) // GB, gather, 0, unroll=False)
        for b in range(GDEPTH):
            pltpu.make_async_copy(
                bt_hbm.at[pl.ds(0, _CHUNK)], pw_v.at[0, pl.ds(0, _CHUNK)], sem
            ).wait()

        n_cblk = S // 128
        rb0 = (h * S + i0) // 8

        def fire_tile(t):
            q = t // n_cblk
            c = t - q * n_cblk
            ib = i0 + 8 * q
            w0 = pl.multiple_of((S - 8) - ib + 128 * c, 8)
            src = pw_v.at[:, pl.ds(w0, 128)]
            dst = out_hbm.at[rb0 + q, c]
            return pltpu.async_copy(src, dst, sem)

        n_tiles = (rows_per_worker // 8) * n_cblk
        DEPTH = 64
        B = 8
        for b in range(DEPTH):
            fire_tile(b)

        def tiles(g, _):
            tb = DEPTH + g * B
            handles = [fire_tile(tb + b) for b in range(B)]
            for hd in handles:
                hd.wait()
            return 0

        lax.fori_loop(0, (n_tiles - DEPTH) // B, tiles, 0, unroll=False)
        for b in range(DEPTH):
            pltpu.make_async_copy(
                out_hbm.at[rb0, 0], pw_v.at[:, pl.ds(0, 128)], sem
            ).wait()

    return sc_kernel


def kernel(x, bias_values):
    S = x.shape[1]
    num_buckets, num_heads = bias_values.shape
    sc = _build_sc_kernel(S, num_buckets, num_heads)
    bt = bias_values.astype(jnp.float32).T.reshape(-1)
    out4 = sc(bt)
    out = (
        out4.reshape(num_heads, S // 8, S // 128, 8, 128)
        .transpose(0, 1, 3, 2, 4)
        .reshape(1, num_heads, S, S)
    )
    return out


# speedup vs baseline: 2.2842x; 2.2842x over previous
import functools

import jax
import jax.numpy as jnp
from jax import lax
from jax.experimental import pallas as pl
from jax.experimental.pallas import tpu as pltpu
from jax.experimental.pallas import tpu_sc as plsc

_LANES = 16
_NUM_CORES = 2
_NUM_SUBCORES = 16
_NUM_WORKERS = _NUM_CORES * _NUM_SUBCORES
_CHUNK = 128
_NVAR = 8


@functools.lru_cache(maxsize=None)
def _build_sc_kernel(S: int, num_buckets: int, num_heads: int):
    L = _LANES
    NW = _NUM_WORKERS
    rows_total = num_heads * S
    assert rows_total % NW == 0
    rows_per_worker = rows_total // NW
    assert rows_per_worker % 64 == 0 and S % rows_per_worker == 0
    assert S % 128 == 0
    P = 2 * S
    assert (_NVAR * P) % (8 * _CHUNK) == 0 and P % _CHUNK == 0
    md = num_buckets // 2
    shift = md - (S - 1)
    G0 = (S - 1) - md - (_NVAR - 1)
    assert G0 % 16 == 0 and G0 > 0
    MIDC = -(-(num_buckets + _NVAR) // _CHUNK)
    MIDW = MIDC * _CHUNK
    R0 = G0 + MIDW
    assert R0 <= P and (P - R0) % 16 == 0 and MIDC % 8 == 0

    mesh = plsc.VectorSubcoreMesh(core_axis_name="c", subcore_axis_name="s")

    @functools.partial(
        pl.kernel,
        mesh=mesh,
        out_type=jax.ShapeDtypeStruct((rows_total // 8, S // 128, 8, 128), jnp.float32),
        compiler_params=pltpu.CompilerParams(use_tc_tiling_on_sc=False),
        scratch_types=[
            pltpu.VMEM((_NVAR * MIDW + 32,), jnp.int32),
            pltpu.VMEM((_NVAR, P), jnp.float32),
            pltpu.VMEM((32,), jnp.float32),
            pltpu.SemaphoreType.DMA,
        ],
    )
    def sc_kernel(bt_hbm, out_hbm, idx_v, pw_v, edge_v, sem):
        wid = lax.axis_index("s") * _NUM_CORES + lax.axis_index("c")
        row0 = wid * rows_per_worker
        h = row0 // S
        i0 = row0 - h * S

        iota = lax.iota(jnp.int32, L)
        hbase = h * num_buckets

        mid_total = _NVAR * MIDW
        idx_v[pl.ds(mid_total, L)] = jnp.full((L,), hbase, jnp.int32)
        idx_v[pl.ds(mid_total + L, L)] = jnp.full(
            (L,), hbase + num_buckets - 1, jnp.int32
        )
        edge_gather = pltpu.async_copy(
            bt_hbm.at[idx_v.at[pl.ds(mid_total, 2 * L)]], edge_v, sem
        )

        def build_idx(slot, _):
            base_e = slot * L
            c0 = base_e + iota
            for k in range(_NVAR):
                idx_v[pl.ds(k * MIDW + base_e, L)] = hbase + jnp.clip(
                    c0 - k, 0, num_buckets - 1
                )
            return 0

        lax.fori_loop(0, MIDW // L, build_idx, 0, unroll=False)
        edge_gather.wait()
        v_lo = edge_v[pl.ds(0, L)]
        v_hi = edge_v[pl.ds(L, L)]

        n_chunks = _NVAR * MIDC

        def fire_chunk(c):
            k = c // MIDC
            off = (c - k * MIDC) * _CHUNK
            src = bt_hbm.at[idx_v.at[pl.ds(k * MIDW + off, _CHUNK)]]
            return pltpu.async_copy(src, pw_v.at[k, pl.ds(G0 + off, _CHUNK)], sem)

        GDEPTH = 32
        GB = 8
        for b in range(GDEPTH):
            fire_chunk(b)

        def gather(g, _):
            cb = GDEPTH + g * GB
            handles = [fire_chunk(cb + b) for b in range(GB)]
            for hd in handles:
                hd.wait()
            return 0

        lax.fori_loop(0, (n_chunks - GDEPTH) // GB, gather, 0, unroll=False)

        def fill_lo(g, _):
            for k in range(_NVAR):
                pw_v[k, pl.ds(g * L, L)] = v_lo
            return 0

        def fill_hi(g, _):
            for k in range(_NVAR):
                pw_v[k, pl.ds(R0 + g * L, L)] = v_hi
            return 0

        lax.fori_loop(0, G0 // L, fill_lo, 0, unroll=False)
        lax.fori_loop(0, (P - R0) // L, fill_hi, 0, unroll=False)
        for b in range(GDEPTH):
            pltpu.make_async_copy(
                bt_hbm.at[pl.ds(0, _CHUNK)], pw_v.at[0, pl.ds(0, _CHUNK)], sem
            ).wait()

        n_cblk = S // 128
        rb0 = (h * S + i0) // 8

        def fire_tile(t):
            q = t // n_cblk
            c = t - q * n_cblk
            ib = i0 + 8 * q
            w0 = pl.multiple_of((S - 8) - ib + 128 * c, 8)
            src = pw_v.at[:, pl.ds(w0, 128)]
            dst = out_hbm.at[rb0 + q, c]
            return pltpu.async_copy(src, dst, sem)

        n_tiles = (rows_per_worker // 8) * n_cblk
        DEPTH = 32
        B = 8
        for b in range(DEPTH):
            fire_tile(b)

        def tiles(g, _):
            tb = DEPTH + g * B
            handles = [fire_tile(tb + b) for b in range(B)]
            for hd in handles:
                hd.wait()
            return 0

        lax.fori_loop(0, (n_tiles - DEPTH) // B, tiles, 0, unroll=False)
        for b in range(DEPTH):
            pltpu.make_async_copy(
                out_hbm.at[rb0, 0], pw_v.at[:, pl.ds(0, 128)], sem
            ).wait()

    return sc_kernel


def kernel(x, bias_values):
    S = x.shape[1]
    num_buckets, num_heads = bias_values.shape
    sc = _build_sc_kernel(S, num_buckets, num_heads)
    bt = bias_values.astype(jnp.float32).T.reshape(-1)
    out4 = sc(bt)
    out = (
        out4.reshape(num_heads, S // 8, S // 128, 8, 128)
        .transpose(0, 1, 3, 2, 4)
        .reshape(1, num_heads, S, S)
    )
    return out
